# Initial kernel scaffold; baseline (speedup 1.0000x reference)
import jax
import jax.numpy as jnp
from jax.experimental import pallas as pl
from jax.experimental.pallas import tpu as pltpu
from jax.experimental.pallas import tpu_sc as plsc

BATCH = 16384
FIELDS = 26
DIM = 32
NUM_IDX = BATCH * FIELDS
WINDOW = 128


def kernel(x, embedding):
    idx = x.reshape(1, NUM_IDX).astype(jnp.int32)
    mesh = plsc.VectorSubcoreMesh(
        core_axis_name="core", subcore_axis_name="subcore"
    )

    @pl.kernel(
        out_type=jax.ShapeDtypeStruct((NUM_IDX, DIM), embedding.dtype),
        mesh=mesh,
    )
    def gather_kernel(emb_hbm, i_hbm, o_hbm):
        def body(i_vmem, o_vmem):
            pltpu.sync_copy(emb_hbm.at[i_vmem.at[0]], o_vmem)

        pltpu.emit_pipeline(
            body,
            grid=(NUM_IDX // WINDOW,),
            in_specs=[
                pl.BlockSpec((1, WINDOW), index_map=lambda i: (0, i))
            ],
            out_specs=[
                pl.BlockSpec((WINDOW, DIM), index_map=lambda i: (i, 0))
            ],
            core_axis_name=("core", "subcore"),
            dimension_semantics=(pltpu.PARALLEL,),
        )(i_hbm, o_hbm)

    out = gather_kernel(embedding, idx)
    return out.reshape(BATCH, FIELDS, DIM)


# speedup vs baseline: 1.5472x; 1.5472x over previous
import functools

import jax
import jax.numpy as jnp
from jax import lax
from jax.experimental import pallas as pl
from jax.experimental.pallas import tpu as pltpu
from jax.experimental.pallas import tpu_sc as plsc

BATCH = 16384
FIELDS = 26
DIM = 32
NUM_IDX = BATCH * FIELDS

NC = 2
NS = 16
NW = NC * NS
B_PER_W = NUM_IDX // NW
CHUNK = 1024
NCHUNK = B_PER_W // CHUNK


def kernel(x, embedding):
    idx = x.reshape(NUM_IDX).astype(jnp.int32)
    mesh = plsc.VectorSubcoreMesh(core_axis_name="c", subcore_axis_name="s")

    @functools.partial(
        pl.kernel,
        mesh=mesh,
        out_type=jax.ShapeDtypeStruct((NUM_IDX, DIM), jnp.float32),
        scratch_types=[
            pltpu.VMEM((CHUNK,), jnp.int32),
            pltpu.VMEM((CHUNK, DIM), jnp.float32),
            pltpu.SemaphoreType.DMA,
        ],
        compiler_params=pltpu.CompilerParams(use_tc_tiling_on_sc=False),
    )
    def gather_kernel(table_hbm, idx_hbm, out_hbm, idx_v, rows_v, sem):
        wid = lax.axis_index("s") * NC + lax.axis_index("c")
        base = wid * B_PER_W

        @pl.loop(0, NCHUNK)
        def _(i):
            off = base + i * CHUNK
            pltpu.sync_copy(idx_hbm.at[pl.ds(off, CHUNK)], idx_v)
            pltpu.async_copy(table_hbm.at[idx_v], rows_v, sem).wait()
            pltpu.sync_copy(rows_v, out_hbm.at[pl.ds(off, CHUNK)])

    out = gather_kernel(embedding, idx)
    return out.reshape(BATCH, FIELDS, DIM)


# speedup vs baseline: 1.5646x; 1.0112x over previous
import functools

import jax
import jax.numpy as jnp
from jax import lax
from jax.experimental import pallas as pl
from jax.experimental.pallas import tpu as pltpu
from jax.experimental.pallas import tpu_sc as plsc

BATCH = 16384
FIELDS = 26
DIM = 32
NUM_IDX = BATCH * FIELDS

NC = 2
NS = 16
NW = NC * NS
B_PER_W = NUM_IDX // NW
NBUF = 4
CHUNK = 832
NCHUNK = B_PER_W // CHUNK


def kernel(x, embedding):
    idx = x.reshape(NUM_IDX).astype(jnp.int32)
    mesh = plsc.VectorSubcoreMesh(core_axis_name="c", subcore_axis_name="s")

    @functools.partial(
        pl.kernel,
        mesh=mesh,
        out_type=jax.ShapeDtypeStruct((NUM_IDX, DIM), jnp.float32),
        scratch_types=[
            pltpu.VMEM((B_PER_W,), jnp.int32),
            [pltpu.VMEM((CHUNK, DIM), jnp.float32) for _ in range(NBUF)],
            [pltpu.SemaphoreType.DMA for _ in range(NBUF)],
            [pltpu.SemaphoreType.DMA for _ in range(NBUF)],
        ],
        compiler_params=pltpu.CompilerParams(use_tc_tiling_on_sc=False),
    )
    def gather_kernel(table_hbm, idx_hbm, out_hbm, idx_v, bufs, gsems, wsems):
        wid = lax.axis_index("s") * NC + lax.axis_index("c")
        base = wid * B_PER_W
        pltpu.sync_copy(idx_hbm.at[pl.ds(base, B_PER_W)], idx_v)

        def gather(c, b):
            return pltpu.make_async_copy(
                table_hbm.at[idx_v.at[pl.ds(c * CHUNK, CHUNK)]],
                bufs[b], gsems[b])

        def write(c, b):
            return pltpu.make_async_copy(
                bufs[b], out_hbm.at[pl.ds(base + c * CHUNK, CHUNK)], wsems[b])

        for b in range(NBUF):
            gather(b, b).start()

        @pl.loop(0, NCHUNK - NBUF, step=NBUF)
        def _(i):
            for b in range(NBUF):
                gather(i + b, b).wait()
                write(i + b, b).start()
            for b in range(NBUF):
                write(i + b, b).wait()
                gather(i + b + NBUF, b).start()

        for b in range(NBUF):
            gather(NCHUNK - NBUF + b, b).wait()
            write(NCHUNK - NBUF + b, b).start()
        for b in range(NBUF):
            write(NCHUNK - NBUF + b, b).wait()

    out = gather_kernel(embedding, idx)
    return out.reshape(BATCH, FIELDS, DIM)


# speedup vs baseline: 1.5675x; 1.0019x over previous
import functools

import jax
import jax.numpy as jnp
from jax import lax
from jax.experimental import pallas as pl
from jax.experimental.pallas import tpu as pltpu
from jax.experimental.pallas import tpu_sc as plsc

BATCH = 16384
FIELDS = 26
DIM = 32
FPAD = 32
NUM_EMB = 1000000

NC = 2
NS = 16
NW = NC * NS
B_PER_W = BATCH // NW
NBUF = 4
CHUNK_B = 32
NCHUNK = B_PER_W // CHUNK_B


def kernel(x, embedding):
    idx = jnp.pad(x.astype(jnp.int32), ((0, 0), (0, FPAD - FIELDS)))
    idx = idx.reshape(BATCH * FPAD)
    mesh = plsc.VectorSubcoreMesh(core_axis_name="c", subcore_axis_name="s")

    @functools.partial(
        pl.kernel,
        mesh=mesh,
        out_type=jax.ShapeDtypeStruct((BATCH, FIELDS, DIM), jnp.float32),
        scratch_types=[
            pltpu.VMEM((B_PER_W * FPAD,), jnp.int32),
            [pltpu.VMEM((CHUNK_B, FIELDS, DIM), jnp.float32)
             for _ in range(NBUF)],
            [pltpu.SemaphoreType.DMA for _ in range(NBUF)],
            [pltpu.SemaphoreType.DMA for _ in range(NBUF)],
        ],
        compiler_params=pltpu.CompilerParams(use_tc_tiling_on_sc=False),
    )
    def gather_kernel(table_hbm, idx_hbm, out_hbm, idx_v, bufs, gsems, wsems):
        wid = lax.axis_index("s") * NC + lax.axis_index("c")
        base_b = wid * B_PER_W
        pltpu.sync_copy(idx_hbm.at[pl.ds(base_b * FPAD, B_PER_W * FPAD)],
                        idx_v)

        def start_gather(c, b):
            @pl.loop(0, CHUNK_B)
            def _(bi):
                pltpu.async_copy(
                    table_hbm.at[
                        idx_v.at[pl.ds((c * CHUNK_B + bi) * FPAD, FIELDS)]],
                    bufs[b].at[bi], gsems[b])

        def wait_gather(c, b):
            @pl.loop(0, CHUNK_B)
            def _(bi):
                pltpu.make_async_copy(
                    table_hbm.at[
                        idx_v.at[pl.ds((c * CHUNK_B + bi) * FPAD, FIELDS)]],
                    bufs[b].at[bi], gsems[b]).wait()

        def write(c, b):
            return pltpu.make_async_copy(
                bufs[b],
                out_hbm.at[pl.ds(base_b + c * CHUNK_B, CHUNK_B)], wsems[b])

        for b in range(NBUF):
            start_gather(b, b)

        @pl.loop(0, NCHUNK - NBUF, step=NBUF)
        def _(i):
            for b in range(NBUF):
                wait_gather(i + b, b)
                write(i + b, b).start()
            for b in range(NBUF):
                write(i + b, b).wait()
                start_gather(i + b + NBUF, b)

        for b in range(NBUF):
            wait_gather(NCHUNK - NBUF + b, b)
            write(NCHUNK - NBUF + b, b).start()
        for b in range(NBUF):
            write(NCHUNK - NBUF + b, b).wait()

    return gather_kernel(embedding, idx)
